# baseline (speedup 1.0000x reference)
import functools

import jax
import jax.numpy as jnp
from jax import lax
from jax.experimental import pallas as pl
from jax.experimental.pallas import tpu as pltpu
from jax.experimental.pallas import tpu_sc as plsc

NC = 2
NS = 16
NW = NC * NS
L = 16
H = 768
HC = H // L
HW = H // 2
HC2 = H // (2 * L)
R = 32
NBX = 3
NBY = 2
EPS = 1e-12
MASK_HI = jnp.int32(-65536)


def _rsqrt(v):
    i = plsc.bitcast(v, jnp.int32)
    y = plsc.bitcast(jnp.int32(0x5F3759DF) - (i >> 1), jnp.float32)
    for _ in range(3):
        y = y * (1.5 - 0.5 * v * y * y)
    return y


def _make_kernel(n_rows):
    rows_per_w = n_rows // NW
    chunks = rows_per_w // R
    mesh = plsc.VectorSubcoreMesh(
        core_axis_name="c", subcore_axis_name="s",
        num_cores=NC, num_subcores=NS)

    @functools.partial(
        pl.kernel,
        out_type=jax.ShapeDtypeStruct((n_rows, H), jnp.int32),
        mesh=mesh,
        compiler_params=pltpu.CompilerParams(needs_layout_passes=False),
        scratch_types=[
            pltpu.VMEM((rows_per_w,), jnp.int32),
            pltpu.VMEM((NBX, R, H), jnp.int32),
            pltpu.VMEM((NBY, R, H), jnp.float32),
            pltpu.VMEM((R * L,), jnp.float32),
            pltpu.VMEM((R * L,), jnp.float32),
            pltpu.SMEM((R,), jnp.float32),
            pltpu.SMEM((R,), jnp.float32),
            pltpu.VMEM((H,), jnp.float32),
            pltpu.VMEM((H,), jnp.float32),
            pltpu.SemaphoreType.DMA((NBX,)),
            pltpu.SemaphoreType.DMA((NBY,)),
            pltpu.SemaphoreType.DMA((NBX,)),
            pltpu.SemaphoreType.DMA,
        ],
    )
    def kern(emb_hbm, ids_hbm, tab_hbm, gam_hbm, bet_hbm, out_hbm,
             ids_v, x_v, y_v, sp_v, sq_v, a_sm, d_sm, g_v, b_v,
             sem_g, sem_e, sem_o, sem_i):
        wid = lax.axis_index("s") * NC + lax.axis_index("c")
        wbase = wid * rows_per_w
        pltpu.sync_copy(gam_hbm, g_v)
        pltpu.sync_copy(bet_hbm, b_v)
        pltpu.async_copy(ids_hbm.at[pl.ds(wbase, rows_per_w)], ids_v,
                         sem_i).wait()

        def start_loads(c, bx, by):
            idx = ids_v.at[pl.ds(c * R, R)]
            pltpu.async_copy(tab_hbm.at[idx],
                             x_v.at[bx, :, pl.ds(0, HW)], sem_g.at[bx])
            pltpu.async_copy(emb_hbm.at[pl.ds(wbase + c * R, R)],
                             y_v.at[by], sem_e.at[by])

        start_loads(0, 0, 0)

        def chunk_body(c, _):
            bx = lax.rem(c, NBX)
            by = lax.rem(c, NBY)

            idx = ids_v.at[pl.ds(c * R, R)]
            pltpu.make_async_copy(tab_hbm.at[idx],
                                  x_v.at[bx, :, pl.ds(0, HW)],
                                  sem_g.at[bx]).wait()
            pltpu.make_async_copy(emb_hbm.at[pl.ds(wbase + c * R, R)],
                                  y_v.at[by], sem_e.at[by]).wait()

            @pl.when(c + 1 < chunks)
            def _():
                nbx = lax.rem(c + 1, NBX)
                nby = lax.rem(c + 1, NBY)
                @pl.when(c >= 2)
                def _():
                    pltpu.make_async_copy(
                        x_v.at[nbx],
                        out_hbm.at[pl.ds(wbase + (c - 2) * R, R)],
                        sem_o.at[nbx]).wait()
                start_loads(c + 1, nbx, nby)

            RI = 2
            def row_body(q, _):
                r0 = q * RI
                def h_body(m, carry):
                    out = []
                    for i in range(RI):
                        s, ss = carry[2 * i], carry[2 * i + 1]
                        sl0 = pl.ds(m * L, L)
                        sl1 = pl.ds(HW + m * L, L)
                        pw = x_v[bx, r0 + i, sl0]
                        lo = plsc.bitcast(pw << 16, jnp.float32)
                        hi = plsc.bitcast(pw & MASK_HI, jnp.float32)
                        x0 = y_v[by, r0 + i, sl0] + lo
                        x1 = y_v[by, r0 + i, sl1] + hi
                        x_v[bx, r0 + i, sl0] = plsc.bitcast(x0, jnp.int32)
                        x_v[bx, r0 + i, sl1] = plsc.bitcast(x1, jnp.int32)
                        out += [s + x0 + x1, ss + x0 * x0 + x1 * x1]
                    return tuple(out)
                z = jnp.zeros((L,), jnp.float32)
                acc = plsc.parallel_loop(
                    0, HC2, 1, unroll=4, carry=(z,) * (2 * RI))(h_body)
                for i in range(RI):
                    sp_v[pl.ds((r0 + i) * L, L)] = acc[2 * i]
                    sq_v[pl.ds((r0 + i) * L, L)] = acc[2 * i + 1]
                return 0
            lax.fori_loop(0, R // RI, row_body, 0)

            for k in range(R // L):
                rows16 = (lax.iota(jnp.int32, L) + k * L) * L
                s = jnp.zeros((L,), jnp.float32)
                ss = jnp.zeros((L,), jnp.float32)
                for j in range(L):
                    fidx = rows16 + j
                    s = s + plsc.load_gather(sp_v, [fidx])
                    ss = ss + plsc.load_gather(sq_v, [fidx])
                mean = s * (1.0 / H)
                var = ss * (1.0 / H) - mean * mean
                rstd = _rsqrt(var + EPS)
                nmr = -mean * rstd
                for j in range(L):
                    a_sm[k * L + j] = rstd[j]
                    d_sm[k * L + j] = nmr[j]

            def hb(h, _):
                sl = pl.ds(h * L, L)
                g = g_v[sl]
                b = b_v[sl]
                def rb(r):
                    x = plsc.bitcast(x_v[bx, r, sl], jnp.float32)
                    y = (x * a_sm[r] + d_sm[r]) * g + b
                    x_v[bx, r, sl] = plsc.bitcast(y, jnp.int32)
                plsc.parallel_loop(0, R, 1, unroll=8)(rb)
                return 0
            lax.fori_loop(0, HC, hb, 0)

            pltpu.async_copy(x_v.at[bx],
                             out_hbm.at[pl.ds(wbase + c * R, R)],
                             sem_o.at[bx])
            return 0

        lax.fori_loop(0, chunks, chunk_body, 0)

        for j in range(NBX):
            pltpu.make_async_copy(x_v.at[j], out_hbm.at[pl.ds(wbase, R)],
                                  sem_o.at[j]).wait()

    return kern


def kernel(inputs_embeds, position_ids, pos_table, ln_gamma, ln_beta):
    b, s, h = inputs_embeds.shape
    n = b * s
    emb = inputs_embeds.reshape(n, h)
    ids = position_ids.reshape(n).astype(jnp.int32)
    b32 = lax.bitcast_convert_type(pos_table, jnp.int32)

    def _bf16_bits(v):
        return ((v + 0x7FFF + ((v >> 16) & 1)) >> 16) & 0xFFFF

    tab_i32 = _bf16_bits(b32[:, :h // 2]) | (_bf16_bits(b32[:, h // 2:]) << 16)
    out = _make_kernel(n)(emb, ids, tab_i32,
                          ln_gamma.astype(jnp.float32),
                          ln_beta.astype(jnp.float32))
    return lax.bitcast_convert_type(out, jnp.float32).reshape(b, s, h)


# speedup vs baseline: 1.3020x; 1.3020x over previous
import functools

import jax
import jax.numpy as jnp
from jax import lax
from jax.experimental import pallas as pl
from jax.experimental.pallas import tpu as pltpu
from jax.experimental.pallas import tpu_sc as plsc

NC = 2
NS = 16
NW = NC * NS
L = 16
H = 768
HC = H // L
HW = H // 2
HC2 = H // (2 * L)
R = 32
NBX = 3
NBY = 2
EPS = 1e-12
MASK_HI = jnp.int32(-65536)


def _rsqrt(v):
    i = plsc.bitcast(v, jnp.int32)
    y = plsc.bitcast(jnp.int32(0x5F3759DF) - (i >> 1), jnp.float32)
    for _ in range(3):
        y = y * (1.5 - 0.5 * v * y * y)
    return y


def _make_kernel(n_rows):
    rows_per_w = n_rows // NW
    chunks = rows_per_w // R
    mesh = plsc.VectorSubcoreMesh(
        core_axis_name="c", subcore_axis_name="s",
        num_cores=NC, num_subcores=NS)

    @functools.partial(
        pl.kernel,
        out_type=jax.ShapeDtypeStruct((n_rows, H), jnp.float32),
        mesh=mesh,
        compiler_params=pltpu.CompilerParams(needs_layout_passes=False),
        scratch_types=[
            pltpu.VMEM((rows_per_w,), jnp.int32),
            pltpu.VMEM((NBX, R, H), jnp.float32),
            pltpu.VMEM((NBY, R, H), jnp.float32),
            pltpu.VMEM((R * L,), jnp.float32),
            pltpu.VMEM((R * L,), jnp.float32),
            pltpu.SMEM((R,), jnp.float32),
            pltpu.SMEM((R,), jnp.float32),
            pltpu.VMEM((H,), jnp.float32),
            pltpu.VMEM((H,), jnp.float32),
            pltpu.SemaphoreType.DMA((NBX,)),
            pltpu.SemaphoreType.DMA((NBY,)),
            pltpu.SemaphoreType.DMA((NBX,)),
            pltpu.SemaphoreType.DMA,
        ],
    )
    def kern(emb_hbm, ids_hbm, tab_hbm, gam_hbm, bet_hbm, out_hbm,
             ids_v, x_v, y_v, sp_v, sq_v, a_sm, d_sm, g_v, b_v,
             sem_g, sem_e, sem_o, sem_i):
        wid = lax.axis_index("s") * NC + lax.axis_index("c")
        wbase = wid * rows_per_w
        pltpu.sync_copy(gam_hbm, g_v)
        pltpu.sync_copy(bet_hbm, b_v)
        pltpu.async_copy(ids_hbm.at[pl.ds(wbase, rows_per_w)], ids_v,
                         sem_i).wait()

        def start_loads(c, bx, by):
            idx = ids_v.at[pl.ds(c * R, R)]
            pltpu.async_copy(tab_hbm.at[idx],
                             x_v.at[bx, :, pl.ds(0, HW)], sem_g.at[bx])
            pltpu.async_copy(emb_hbm.at[pl.ds(wbase + c * R, R)],
                             y_v.at[by], sem_e.at[by])

        start_loads(0, 0, 0)

        def chunk_body(c, _):
            bx = lax.rem(c, NBX)
            by = lax.rem(c, NBY)

            idx = ids_v.at[pl.ds(c * R, R)]
            pltpu.make_async_copy(tab_hbm.at[idx],
                                  x_v.at[bx, :, pl.ds(0, HW)],
                                  sem_g.at[bx]).wait()
            pltpu.make_async_copy(emb_hbm.at[pl.ds(wbase + c * R, R)],
                                  y_v.at[by], sem_e.at[by]).wait()

            @pl.when(c + 1 < chunks)
            def _():
                nbx = lax.rem(c + 1, NBX)
                nby = lax.rem(c + 1, NBY)
                @pl.when(c >= 2)
                def _():
                    pltpu.make_async_copy(
                        x_v.at[nbx],
                        out_hbm.at[pl.ds(wbase + (c - 2) * R, R)],
                        sem_o.at[nbx]).wait()
                start_loads(c + 1, nbx, nby)

            RI = 2
            def row_body(q, _):
                r0 = q * RI
                def h_body(m, carry):
                    out = []
                    for i in range(RI):
                        s, ss = carry[2 * i], carry[2 * i + 1]
                        sl0 = pl.ds(m * L, L)
                        sl1 = pl.ds(HW + m * L, L)
                        pw = plsc.bitcast(x_v[bx, r0 + i, sl0], jnp.int32)
                        lo = plsc.bitcast(pw << 16, jnp.float32)
                        hi = plsc.bitcast(pw & MASK_HI, jnp.float32)
                        x0 = y_v[by, r0 + i, sl0] + lo
                        x1 = y_v[by, r0 + i, sl1] + hi
                        x_v[bx, r0 + i, sl0] = x0
                        x_v[bx, r0 + i, sl1] = x1
                        out += [s + x0 + x1, ss + x0 * x0 + x1 * x1]
                    return tuple(out)
                z = jnp.zeros((L,), jnp.float32)
                acc = plsc.parallel_loop(
                    0, HC2, 1, unroll=4, carry=(z,) * (2 * RI))(h_body)
                for i in range(RI):
                    sp_v[pl.ds((r0 + i) * L, L)] = acc[2 * i]
                    sq_v[pl.ds((r0 + i) * L, L)] = acc[2 * i + 1]
                return 0
            lax.fori_loop(0, R // RI, row_body, 0)

            for k in range(R // L):
                rows16 = (lax.iota(jnp.int32, L) + k * L) * L
                s = jnp.zeros((L,), jnp.float32)
                ss = jnp.zeros((L,), jnp.float32)
                for j in range(L):
                    fidx = rows16 + j
                    s = s + plsc.load_gather(sp_v, [fidx])
                    ss = ss + plsc.load_gather(sq_v, [fidx])
                mean = s * (1.0 / H)
                var = ss * (1.0 / H) - mean * mean
                rstd = _rsqrt(var + EPS)
                nmr = -mean * rstd
                for j in range(L):
                    a_sm[k * L + j] = rstd[j]
                    d_sm[k * L + j] = nmr[j]

            def hb(h, _):
                sl = pl.ds(h * L, L)
                g = g_v[sl]
                b = b_v[sl]
                def rb(r):
                    x = x_v[bx, r, sl]
                    x_v[bx, r, sl] = (x * a_sm[r] + d_sm[r]) * g + b
                plsc.parallel_loop(0, R, 1, unroll=8)(rb)
                return 0
            lax.fori_loop(0, HC, hb, 0)

            pltpu.async_copy(x_v.at[bx],
                             out_hbm.at[pl.ds(wbase + c * R, R)],
                             sem_o.at[bx])
            return 0

        lax.fori_loop(0, chunks, chunk_body, 0)

        for j in range(NBX):
            pltpu.make_async_copy(x_v.at[j], out_hbm.at[pl.ds(wbase, R)],
                                  sem_o.at[j]).wait()

    return kern


def kernel(inputs_embeds, position_ids, pos_table, ln_gamma, ln_beta):
    b, s, h = inputs_embeds.shape
    n = b * s
    emb = inputs_embeds.reshape(n, h)
    ids = position_ids.reshape(n).astype(jnp.int32)
    b32 = lax.bitcast_convert_type(pos_table, jnp.int32)

    def _bf16_bits(v):
        return ((v + 0x7FFF + ((v >> 16) & 1)) >> 16) & 0xFFFF

    tab_i32 = _bf16_bits(b32[:, :h // 2]) | (_bf16_bits(b32[:, h // 2:]) << 16)
    tab_f32 = lax.bitcast_convert_type(tab_i32, jnp.float32)
    out = _make_kernel(n)(emb, ids, tab_f32,
                          ln_gamma.astype(jnp.float32),
                          ln_beta.astype(jnp.float32))
    return out.reshape(b, s, h)


# speedup vs baseline: 1.4938x; 1.1473x over previous
import functools

import jax
import jax.numpy as jnp
from jax import lax
from jax.experimental import pallas as pl
from jax.experimental.pallas import tpu as pltpu
from jax.experimental.pallas import tpu_sc as plsc

NC = 2
NS = 16
NW = NC * NS
L = 16
H = 768
HC = H // L
HW = H // 2
HC2 = H // (2 * L)
R = 32
NBX = 3
NBP = 2
EPS = 1e-12
MASK_HI = jnp.int32(-65536)


def _rsqrt(v):
    i = plsc.bitcast(v, jnp.int32)
    y = plsc.bitcast(jnp.int32(0x5F3759DF) - (i >> 1), jnp.float32)
    for _ in range(3):
        y = y * (1.5 - 0.5 * v * y * y)
    return y


def _make_kernel(n_rows):
    rows_per_w = n_rows // NW
    chunks = rows_per_w // R
    mesh = plsc.VectorSubcoreMesh(
        core_axis_name="c", subcore_axis_name="s",
        num_cores=NC, num_subcores=NS)

    @functools.partial(
        pl.kernel,
        out_type=jax.ShapeDtypeStruct((n_rows, H), jnp.float32),
        mesh=mesh,
        compiler_params=pltpu.CompilerParams(needs_layout_passes=False),
        scratch_types=[
            pltpu.VMEM((rows_per_w,), jnp.int32),
            pltpu.VMEM((NBP, R, HW), jnp.int32),
            pltpu.VMEM((NBX, R, H), jnp.float32),
            pltpu.VMEM((R * L,), jnp.float32),
            pltpu.VMEM((R * L,), jnp.float32),
            pltpu.SMEM((R,), jnp.float32),
            pltpu.SMEM((R,), jnp.float32),
            pltpu.VMEM((H,), jnp.float32),
            pltpu.VMEM((H,), jnp.float32),
            pltpu.SemaphoreType.DMA((NBP,)),
            pltpu.SemaphoreType.DMA((NBX,)),
            pltpu.SemaphoreType.DMA((NBX,)),
            pltpu.SemaphoreType.DMA,
        ],
    )
    def kern(emb_hbm, ids_hbm, tab_hbm, gam_hbm, bet_hbm, out_hbm,
             ids_v, p_v, x_v, sp_v, sq_v, a_sm, d_sm, g_v, b_v,
             sem_g, sem_e, sem_o, sem_i):
        wid = lax.axis_index("s") * NC + lax.axis_index("c")
        wbase = wid * rows_per_w
        pltpu.sync_copy(gam_hbm, g_v)
        pltpu.sync_copy(bet_hbm, b_v)
        pltpu.async_copy(ids_hbm.at[pl.ds(wbase, rows_per_w)], ids_v,
                         sem_i).wait()

        def start_loads(c, bp, bx):
            idx = ids_v.at[pl.ds(c * R, R)]
            pltpu.async_copy(tab_hbm.at[idx], p_v.at[bp], sem_g.at[bp])
            pltpu.async_copy(emb_hbm.at[pl.ds(wbase + c * R, R)],
                             x_v.at[bx], sem_e.at[bx])

        start_loads(0, 0, 0)

        def chunk_body(c, _):
            bp = lax.rem(c, NBP)
            bx = lax.rem(c, NBX)

            idx = ids_v.at[pl.ds(c * R, R)]
            pltpu.make_async_copy(tab_hbm.at[idx], p_v.at[bp],
                                  sem_g.at[bp]).wait()
            pltpu.make_async_copy(emb_hbm.at[pl.ds(wbase + c * R, R)],
                                  x_v.at[bx], sem_e.at[bx]).wait()

            @pl.when(c + 1 < chunks)
            def _():
                nbp = lax.rem(c + 1, NBP)
                nbx = lax.rem(c + 1, NBX)
                @pl.when(c >= 2)
                def _():
                    pltpu.make_async_copy(
                        x_v.at[nbx],
                        out_hbm.at[pl.ds(wbase + (c - 2) * R, R)],
                        sem_o.at[nbx]).wait()
                start_loads(c + 1, nbp, nbx)

            RI = 2
            def row_body(q, _):
                r0 = q * RI
                def h_body(m, carry):
                    out = []
                    for i in range(RI):
                        s, ss = carry[2 * i], carry[2 * i + 1]
                        sl0 = pl.ds(m * L, L)
                        sl1 = pl.ds(HW + m * L, L)
                        pw = p_v[bp, r0 + i, sl0]
                        lo = plsc.bitcast(pw << 16, jnp.float32)
                        hi = plsc.bitcast(pw & MASK_HI, jnp.float32)
                        x0 = x_v[bx, r0 + i, sl0] + lo
                        x1 = x_v[bx, r0 + i, sl1] + hi
                        x_v[bx, r0 + i, sl0] = x0
                        x_v[bx, r0 + i, sl1] = x1
                        out += [s + x0 + x1, ss + x0 * x0 + x1 * x1]
                    return tuple(out)
                z = jnp.zeros((L,), jnp.float32)
                acc = plsc.parallel_loop(
                    0, HC2, 1, unroll=4, carry=(z,) * (2 * RI))(h_body)
                for i in range(RI):
                    sp_v[pl.ds((r0 + i) * L, L)] = acc[2 * i]
                    sq_v[pl.ds((r0 + i) * L, L)] = acc[2 * i + 1]
                return 0
            lax.fori_loop(0, R // RI, row_body, 0)

            for k in range(R // L):
                rows16 = (lax.iota(jnp.int32, L) + k * L) * L
                s = jnp.zeros((L,), jnp.float32)
                ss = jnp.zeros((L,), jnp.float32)
                for j in range(L):
                    fidx = rows16 + j
                    s = s + plsc.load_gather(sp_v, [fidx])
                    ss = ss + plsc.load_gather(sq_v, [fidx])
                mean = s * (1.0 / H)
                var = ss * (1.0 / H) - mean * mean
                rstd = _rsqrt(var + EPS)
                nmr = -mean * rstd
                for j in range(L):
                    a_sm[k * L + j] = rstd[j]
                    d_sm[k * L + j] = nmr[j]

            def hb(h, _):
                sl = pl.ds(h * L, L)
                g = g_v[sl]
                b = b_v[sl]
                def rb(r):
                    x = x_v[bx, r, sl]
                    x_v[bx, r, sl] = (x * a_sm[r] + d_sm[r]) * g + b
                plsc.parallel_loop(0, R, 1, unroll=8)(rb)
                return 0
            lax.fori_loop(0, HC, hb, 0)

            pltpu.async_copy(x_v.at[bx],
                             out_hbm.at[pl.ds(wbase + c * R, R)],
                             sem_o.at[bx])
            return 0

        lax.fori_loop(0, chunks, chunk_body, 0)

        for j in range(NBX):
            pltpu.make_async_copy(x_v.at[j], out_hbm.at[pl.ds(wbase, R)],
                                  sem_o.at[j]).wait()

    return kern


def kernel(inputs_embeds, position_ids, pos_table, ln_gamma, ln_beta):
    b, s, h = inputs_embeds.shape
    n = b * s
    emb = inputs_embeds.reshape(n, h)
    ids = position_ids.reshape(n).astype(jnp.int32)
    b32 = lax.bitcast_convert_type(pos_table, jnp.int32)

    def _bf16_bits(v):
        return ((v + 0x7FFF + ((v >> 16) & 1)) >> 16) & 0xFFFF

    tab_i32 = _bf16_bits(b32[:, :h // 2]) | (_bf16_bits(b32[:, h // 2:]) << 16)
    out = _make_kernel(n)(emb, ids, tab_i32,
                          ln_gamma.astype(jnp.float32),
                          ln_beta.astype(jnp.float32))
    return out.reshape(b, s, h)


# speedup vs baseline: 1.6080x; 1.0765x over previous
import functools

import jax
import jax.numpy as jnp
from jax import lax
from jax.experimental import pallas as pl
from jax.experimental.pallas import tpu as pltpu
from jax.experimental.pallas import tpu_sc as plsc

NC = 2
NS = 16
NW = NC * NS
L = 16
H = 768
HC = H // L
R = 32
NBX = 3
NBY = 2
EPS = 1e-12


def _rsqrt(v):
    i = plsc.bitcast(v, jnp.int32)
    y = plsc.bitcast(jnp.int32(0x5F3759DF) - (i >> 1), jnp.float32)
    for _ in range(3):
        y = y * (1.5 - 0.5 * v * y * y)
    return y


def _make_kernel(n_rows):
    rows_per_w = n_rows // NW
    chunks = rows_per_w // R
    mesh = plsc.VectorSubcoreMesh(
        core_axis_name="c", subcore_axis_name="s",
        num_cores=NC, num_subcores=NS)

    @functools.partial(
        pl.kernel,
        out_type=jax.ShapeDtypeStruct((n_rows, H), jnp.float32),
        mesh=mesh,
        compiler_params=pltpu.CompilerParams(needs_layout_passes=False),
        scratch_types=[
            pltpu.VMEM((rows_per_w,), jnp.int32),
            pltpu.VMEM((NBX, R, H), jnp.float32),
            pltpu.VMEM((NBY, R, H), jnp.float32),
            pltpu.VMEM((R * L,), jnp.float32),
            pltpu.VMEM((R * L,), jnp.float32),
            pltpu.SMEM((R,), jnp.float32),
            pltpu.SMEM((R,), jnp.float32),
            pltpu.VMEM((H,), jnp.float32),
            pltpu.VMEM((H,), jnp.float32),
            pltpu.SemaphoreType.DMA((NBX,)),
            pltpu.SemaphoreType.DMA((NBY,)),
            pltpu.SemaphoreType.DMA((NBX,)),
            pltpu.SemaphoreType.DMA,
        ],
    )
    def kern(emb_hbm, ids_hbm, tab_hbm, gam_hbm, bet_hbm, out_hbm,
             ids_v, x_v, y_v, sp_v, sq_v, a_sm, d_sm, g_v, b_v,
             sem_g, sem_e, sem_o, sem_i):
        wid = lax.axis_index("s") * NC + lax.axis_index("c")
        wbase = wid * rows_per_w
        pltpu.sync_copy(gam_hbm, g_v)
        pltpu.sync_copy(bet_hbm, b_v)
        pltpu.async_copy(ids_hbm.at[pl.ds(wbase, rows_per_w)], ids_v,
                         sem_i).wait()

        def start_loads(c, bx, by):
            idx = ids_v.at[pl.ds(c * R, R)]
            pltpu.async_copy(tab_hbm.at[idx], x_v.at[bx], sem_g.at[bx])
            pltpu.async_copy(emb_hbm.at[pl.ds(wbase + c * R, R)],
                             y_v.at[by], sem_e.at[by])

        start_loads(0, 0, 0)

        def chunk_body(c, _):
            bx = lax.rem(c, NBX)
            by = lax.rem(c, NBY)

            idx = ids_v.at[pl.ds(c * R, R)]
            pltpu.make_async_copy(tab_hbm.at[idx], x_v.at[bx],
                                  sem_g.at[bx]).wait()
            pltpu.make_async_copy(emb_hbm.at[pl.ds(wbase + c * R, R)],
                                  y_v.at[by], sem_e.at[by]).wait()

            @pl.when(c + 1 < chunks)
            def _():
                nbx = lax.rem(c + 1, NBX)
                nby = lax.rem(c + 1, NBY)
                @pl.when(c >= 2)
                def _():
                    pltpu.make_async_copy(
                        x_v.at[nbx],
                        out_hbm.at[pl.ds(wbase + (c - 2) * R, R)],
                        sem_o.at[nbx]).wait()
                start_loads(c + 1, nbx, nby)

            RI = 4
            def row_body(q, _):
                r0 = q * RI
                def h_body(h, carry):
                    sl = pl.ds(h * L, L)
                    out = []
                    for i in range(RI):
                        s, ss = carry[2 * i], carry[2 * i + 1]
                        x = x_v[bx, r0 + i, sl] + y_v[by, r0 + i, sl]
                        x_v[bx, r0 + i, sl] = x
                        out += [s + x, ss + x * x]
                    return tuple(out)
                z = jnp.zeros((L,), jnp.float32)
                acc = plsc.parallel_loop(
                    0, HC, 1, unroll=4, carry=(z,) * (2 * RI))(h_body)
                for i in range(RI):
                    sp_v[pl.ds((r0 + i) * L, L)] = acc[2 * i]
                    sq_v[pl.ds((r0 + i) * L, L)] = acc[2 * i + 1]
                return 0
            lax.fori_loop(0, R // RI, row_body, 0)

            for k in range(R // L):
                rows16 = (lax.iota(jnp.int32, L) + k * L) * L
                s = jnp.zeros((L,), jnp.float32)
                ss = jnp.zeros((L,), jnp.float32)
                for j in range(L):
                    fidx = rows16 + j
                    s = s + plsc.load_gather(sp_v, [fidx])
                    ss = ss + plsc.load_gather(sq_v, [fidx])
                mean = s * (1.0 / H)
                var = ss * (1.0 / H) - mean * mean
                rstd = _rsqrt(var + EPS)
                nmr = -mean * rstd
                for j in range(L):
                    a_sm[k * L + j] = rstd[j]
                    d_sm[k * L + j] = nmr[j]

            def hb(h, _):
                sl = pl.ds(h * L, L)
                g = g_v[sl]
                b = b_v[sl]
                def rb(r):
                    x = x_v[bx, r, sl]
                    x_v[bx, r, sl] = (x * a_sm[r] + d_sm[r]) * g + b
                plsc.parallel_loop(0, R, 1, unroll=8)(rb)
                return 0
            lax.fori_loop(0, HC, hb, 0)

            pltpu.async_copy(x_v.at[bx],
                             out_hbm.at[pl.ds(wbase + c * R, R)],
                             sem_o.at[bx])
            return 0

        lax.fori_loop(0, chunks, chunk_body, 0)

        for j in range(NBX):
            pltpu.make_async_copy(x_v.at[j], out_hbm.at[pl.ds(wbase, R)],
                                  sem_o.at[j]).wait()

    return kern


def kernel(inputs_embeds, position_ids, pos_table, ln_gamma, ln_beta):
    b, s, h = inputs_embeds.shape
    n = b * s
    emb = inputs_embeds.reshape(n, h)
    ids = position_ids.reshape(n).astype(jnp.int32)
    out = _make_kernel(n)(emb, ids, pos_table,
                          ln_gamma.astype(jnp.float32),
                          ln_beta.astype(jnp.float32))
    return out.reshape(b, s, h)


# speedup vs baseline: 1.6091x; 1.0007x over previous
import functools

import jax
import jax.numpy as jnp
from jax import lax
from jax.experimental import pallas as pl
from jax.experimental.pallas import tpu as pltpu
from jax.experimental.pallas import tpu_sc as plsc

NC = 2
NS = 16
NW = NC * NS
L = 16
H = 768
HC = H // L
R = 32
NBX = 3
NBY = 2
EPS = 1e-12


def _rsqrt(v):
    i = plsc.bitcast(v, jnp.int32)
    y = plsc.bitcast(jnp.int32(0x5F3759DF) - (i >> 1), jnp.float32)
    for _ in range(3):
        y = y * (1.5 - 0.5 * v * y * y)
    return y


def _make_kernel(n_rows):
    rows_per_w = n_rows // NW
    chunks = rows_per_w // R
    mesh = plsc.VectorSubcoreMesh(
        core_axis_name="c", subcore_axis_name="s",
        num_cores=NC, num_subcores=NS)

    @functools.partial(
        pl.kernel,
        out_type=jax.ShapeDtypeStruct((n_rows, H), jnp.float32),
        mesh=mesh,
        compiler_params=pltpu.CompilerParams(needs_layout_passes=False),
        scratch_types=[
            pltpu.VMEM((rows_per_w,), jnp.int32),
            pltpu.VMEM((NBX, R, H), jnp.float32),
            pltpu.VMEM((NBY, R, H), jnp.float32),
            pltpu.VMEM((R * L,), jnp.float32),
            pltpu.VMEM((R * L,), jnp.float32),
            pltpu.SMEM((R,), jnp.float32),
            pltpu.SMEM((R,), jnp.float32),
            pltpu.VMEM((H,), jnp.float32),
            pltpu.VMEM((H,), jnp.float32),
            pltpu.SemaphoreType.DMA((NBX,)),
            pltpu.SemaphoreType.DMA((NBY,)),
            pltpu.SemaphoreType.DMA((NBX,)),
            pltpu.SemaphoreType.DMA,
        ],
    )
    def kern(emb_hbm, ids_hbm, tab_hbm, gam_hbm, bet_hbm, out_hbm,
             ids_v, x_v, y_v, sp_v, sq_v, a_sm, d_sm, g_v, b_v,
             sem_g, sem_e, sem_o, sem_i):
        wid = lax.axis_index("s") * NC + lax.axis_index("c")
        wbase = wid * rows_per_w
        pltpu.sync_copy(gam_hbm, g_v)
        pltpu.sync_copy(bet_hbm, b_v)
        pltpu.async_copy(ids_hbm.at[pl.ds(wbase, rows_per_w)], ids_v,
                         sem_i).wait()

        def start_loads(c, bx, by):
            idx = ids_v.at[pl.ds(c * R, R)]
            pltpu.async_copy(tab_hbm.at[idx], x_v.at[bx], sem_g.at[bx])
            pltpu.async_copy(emb_hbm.at[pl.ds(wbase + c * R, R)],
                             y_v.at[by], sem_e.at[by])

        start_loads(0, 0, 0)

        def chunk_body(c, _):
            bx = lax.rem(c, NBX)
            by = lax.rem(c, NBY)

            idx = ids_v.at[pl.ds(c * R, R)]
            pltpu.make_async_copy(tab_hbm.at[idx], x_v.at[bx],
                                  sem_g.at[bx]).wait()
            pltpu.make_async_copy(emb_hbm.at[pl.ds(wbase + c * R, R)],
                                  y_v.at[by], sem_e.at[by]).wait()

            @pl.when(c + 1 < chunks)
            def _():
                nbx = lax.rem(c + 1, NBX)
                nby = lax.rem(c + 1, NBY)
                pltpu.async_copy(
                    emb_hbm.at[pl.ds(wbase + (c + 1) * R, R)],
                    y_v.at[nby], sem_e.at[nby])
                @pl.when(c >= 2)
                def _():
                    pltpu.make_async_copy(
                        x_v.at[nbx],
                        out_hbm.at[pl.ds(wbase + (c - 2) * R, R)],
                        sem_o.at[nbx]).wait()
                pltpu.async_copy(tab_hbm.at[ids_v.at[pl.ds((c + 1) * R, R)]],
                                 x_v.at[nbx], sem_g.at[nbx])

            RI = 4
            def row_body(q, _):
                r0 = q * RI
                def h_body(h, carry):
                    sl = pl.ds(h * L, L)
                    out = []
                    for i in range(RI):
                        s, ss = carry[2 * i], carry[2 * i + 1]
                        x = x_v[bx, r0 + i, sl] + y_v[by, r0 + i, sl]
                        x_v[bx, r0 + i, sl] = x
                        out += [s + x, ss + x * x]
                    return tuple(out)
                z = jnp.zeros((L,), jnp.float32)
                acc = plsc.parallel_loop(
                    0, HC, 1, unroll=4, carry=(z,) * (2 * RI))(h_body)
                for i in range(RI):
                    sp_v[pl.ds((r0 + i) * L, L)] = acc[2 * i]
                    sq_v[pl.ds((r0 + i) * L, L)] = acc[2 * i + 1]
                return 0
            lax.fori_loop(0, R // RI, row_body, 0)

            for k in range(R // L):
                rows16 = (lax.iota(jnp.int32, L) + k * L) * L
                s = jnp.zeros((L,), jnp.float32)
                ss = jnp.zeros((L,), jnp.float32)
                for j in range(L):
                    fidx = rows16 + j
                    s = s + plsc.load_gather(sp_v, [fidx])
                    ss = ss + plsc.load_gather(sq_v, [fidx])
                mean = s * (1.0 / H)
                var = ss * (1.0 / H) - mean * mean
                rstd = _rsqrt(var + EPS)
                nmr = -mean * rstd
                for j in range(L):
                    a_sm[k * L + j] = rstd[j]
                    d_sm[k * L + j] = nmr[j]

            def hb(h, _):
                sl = pl.ds(h * L, L)
                g = g_v[sl]
                b = b_v[sl]
                def rb(r):
                    x = x_v[bx, r, sl]
                    x_v[bx, r, sl] = (x * a_sm[r] + d_sm[r]) * g + b
                plsc.parallel_loop(0, R, 1, unroll=8)(rb)
                return 0
            lax.fori_loop(0, HC, hb, 0)

            pltpu.async_copy(x_v.at[bx],
                             out_hbm.at[pl.ds(wbase + c * R, R)],
                             sem_o.at[bx])
            return 0

        lax.fori_loop(0, chunks, chunk_body, 0)

        for j in range(NBX):
            pltpu.make_async_copy(x_v.at[j], out_hbm.at[pl.ds(wbase, R)],
                                  sem_o.at[j]).wait()

    return kern


def kernel(inputs_embeds, position_ids, pos_table, ln_gamma, ln_beta):
    b, s, h = inputs_embeds.shape
    n = b * s
    emb = inputs_embeds.reshape(n, h)
    ids = position_ids.reshape(n).astype(jnp.int32)
    out = _make_kernel(n)(emb, ids, pos_table,
                          ln_gamma.astype(jnp.float32),
                          ln_beta.astype(jnp.float32))
    return out.reshape(b, s, h)
